# baseline (speedup 1.0000x reference)
import functools

import jax
import jax.numpy as jnp
from jax import lax
from jax.experimental import pallas as pl
from jax.experimental.pallas import tpu as pltpu
from jax.experimental.pallas import tpu_sc as plsc

N = 10000
E = 320000
D = 128
H = 128
O = 16
R = 3
B = 4096

NP = 10240
NC = 2
NS = 16
NW = NC * NS
EPW = E // NW
K = 80
GMAX = EPW // K
GPAD = 128

_mesh = functools.partial(
    plsc.VectorSubcoreMesh, core_axis_name="c", subcore_axis_name="s",
    num_cores=NC, num_subcores=NS)

f32 = jnp.float32
i32 = jnp.int32


def _wid():
    return lax.axis_index("s") * NC + lax.axis_index("c")


def _sc_degrees(s0, d0, s1, d1, s2, d2, onesK, zNP):
    @functools.partial(
        pl.kernel,
        out_type=jax.ShapeDtypeStruct((NC, 6, NP, O), f32),
        mesh=_mesh(),
        scratch_types=[
            [pltpu.VMEM_SHARED((NP, O), f32) for _ in range(3)],
            [pltpu.VMEM((GPAD, K), i32) for _ in range(3)],
            pltpu.VMEM((K, O), f32),
            pltpu.VMEM((640, O), f32),
            [pltpu.SemaphoreType.DMA for _ in range(3)],
        ],
        compiler_params=pltpu.CompilerParams(use_tc_tiling_on_sc=False),
    )
    def k(s0h, d0h, s1h, d1h, s2h, d2h, ones_h, z_h, degp, accs, idxs,
          ones_v, buf_v, sems):
        c = lax.axis_index("c")
        s = lax.axis_index("s")
        wid = _wid()
        pltpu.sync_copy(ones_h, ones_v)
        pltpu.sync_copy(z_h.at[pl.ds(0, 640), :], buf_v)
        es = [s0h, d0h, s1h, d1h, s2h, d2h]
        for grp in range(2):
            for jj in range(3):
                pltpu.sync_copy(buf_v, accs[jj].at[pl.ds(s * 640, 640), :])
                pltpu.sync_copy(es[grp * 3 + jj].at[wid], idxs[jj])
            plsc.subcore_barrier()

            def body(g, _):
                cps = [pltpu.async_copy(ones_v,
                                        accs[jj].at[idxs[jj].at[g]],
                                        sems[jj], add=True)
                       for jj in range(3)]
                for cp in cps:
                    cp.wait()
                return 0

            lax.fori_loop(0, GMAX, body, 0)
            plsc.subcore_barrier()
            for jj in range(3):
                pltpu.sync_copy(accs[jj].at[pl.ds(s * 640, 640), :], buf_v)
                pltpu.sync_copy(buf_v,
                                degp.at[c, grp * 3 + jj,
                                        pl.ds(s * 640, 640), :])
            pltpu.sync_copy(z_h.at[pl.ds(0, 640), :], buf_v)
            plsc.subcore_barrier()

    return k(s0, d0, s1, d1, s2, d2, onesK, zNP)


def _tc_matmul1(features, degp, W1, b1):
    NB = 2000

    def body(x_ref, degp_ref, w_ref, b_ref, h1_ref, h1s_ref):
        ds = degp_ref[0, 0, 0, :, 0] + degp_ref[1, 0, 0, :, 0]
        rs = jnp.where(ds > 0, lax.rsqrt(jnp.maximum(ds, 1e-20)), 0.0)
        h1 = lax.dot_general(x_ref[...], w_ref[0],
                             (((1,), (0,)), ((), ())),
                             preferred_element_type=f32) + b_ref[0, 0]
        h1_ref[0] = h1
        h1s = h1 * rs[:, None]
        h1s_ref[0, 0] = h1s[:, :64]
        h1s_ref[0, 1] = h1s[:, 64:]

    return pl.pallas_call(
        body,
        grid=(R, N // NB),
        in_specs=[
            pl.BlockSpec((NB, D), lambda r, n: (n, 0)),
            pl.BlockSpec((NC, 1, 2, NB, O), lambda r, n: (0, r, 0, n, 0)),
            pl.BlockSpec((1, D, H), lambda r, n: (r, 0, 0)),
            pl.BlockSpec((1, 1, H), lambda r, n: (r, 0, 0)),
        ],
        out_specs=[
            pl.BlockSpec((1, NB, H), lambda r, n: (r, n, 0)),
            pl.BlockSpec((1, 2, NB, 64), lambda r, n: (r, 0, n, 0)),
        ],
        out_shape=[
            jax.ShapeDtypeStruct((R, N, H), f32),
            jax.ShapeDtypeStruct((R, 2, N, 64), f32),
        ],
    )(features, degp, W1, b1.reshape(R, 1, H))


def _sc_segsum(hsplit, s0, d0, s1, d1, s2, d2, zNH):
    HH = H // 2

    @functools.partial(
        pl.kernel,
        out_type=jax.ShapeDtypeStruct((NC, R, 2, NP, HH), f32),
        mesh=_mesh(),
        scratch_types=[
            pltpu.VMEM_SHARED((NP, HH), f32),
            pltpu.VMEM((GPAD, K), i32),
            pltpu.VMEM((GPAD, K), i32),
            [pltpu.VMEM((K, HH), f32) for _ in range(4)],
            pltpu.VMEM((64, HH), f32),
            [pltpu.SemaphoreType.DMA for _ in range(4)],
            [pltpu.SemaphoreType.DMA for _ in range(4)],
        ],
        compiler_params=pltpu.CompilerParams(use_tc_tiling_on_sc=False),
    )
    def k(hs00h, hs01h, hs10h, hs11h, hs20h, hs21h, s0h, d0h, s1h, d1h,
          s2h, d2h, z_h, aggp, acc, ixs, ixd, bufs, zbuf, semG, semW):
        c = lax.axis_index("c")
        s = lax.axis_index("s")
        wid = _wid()
        hss = [[hs00h, hs01h], [hs10h, hs11h], [hs20h, hs21h]]
        ess = [(s0h, d0h), (s1h, d1h), (s2h, d2h)]
        pltpu.sync_copy(z_h.at[pl.ds(0, 64), :], zbuf)
        for r in range(R):
            sh, dh = ess[r]
            pltpu.sync_copy(sh.at[wid], ixs)
            pltpu.sync_copy(dh.at[wid], ixd)
            for hh in range(2):
                for j in range(10):
                    pltpu.sync_copy(zbuf,
                                    acc.at[pl.ds(s * 640 + j * 64, 64), :])
                plsc.subcore_barrier()
                tbl = hss[r][hh]
                for u in range(4):
                    pltpu.async_copy(tbl.at[ixs.at[u]], bufs[u], semG[u])

                def body(t, _):
                    scats = []
                    for u in range(4):
                        j = 4 * t + u
                        pltpu.make_async_copy(tbl.at[ixs.at[j]], bufs[u],
                                              semG[u]).wait()
                        scats.append(pltpu.async_copy(
                            bufs[u], acc.at[ixd.at[j]], semW[u], add=True))
                    for u in range(4):
                        scats[u].wait()
                        pltpu.async_copy(tbl.at[ixs.at[4 * t + u + 4]],
                                         bufs[u], semG[u])
                    return 0

                lax.fori_loop(0, GMAX // 4, body, 0)
                pltpu.make_async_copy(tbl.at[ixs.at[GMAX - 1]], bufs[0],
                                      semG[0]).wait()
                pltpu.sync_copy(bufs[0], acc.at[ixd.at[GMAX - 1]],
                                add=True)
                for u in range(1, 4):
                    pltpu.make_async_copy(tbl.at[ixs.at[GMAX - 1]],
                                          bufs[u], semG[u]).wait()
                plsc.subcore_barrier()
                for j in range(8):
                    pltpu.sync_copy(acc.at[pl.ds(s * 640 + j * K, K), :],
                                    bufs[0])
                    pltpu.sync_copy(
                        bufs[0],
                        aggp.at[c, r, hh, pl.ds(s * 640 + j * K, K), :])
                plsc.subcore_barrier()

    return k(hsplit[0][0], hsplit[0][1], hsplit[1][0], hsplit[1][1],
             hsplit[2][0], hsplit[2][1], s0, d0, s1, d1, s2, d2, zNH)


def _tc_mix(aggp, h1, degp, gamma1):
    NB = 2000

    def body(g_ref, aggp_ref, h1_ref, degp_ref, hmix_ref):
        dd = degp_ref[0, 0, 1, :, 0] + degp_ref[1, 0, 1, :, 0]
        rd = jnp.where(dd > 0, lax.rsqrt(jnp.maximum(dd, 1e-20)), 0.0)
        agg = jnp.concatenate(
            [aggp_ref[0, 0, 0] + aggp_ref[1, 0, 0],
             aggp_ref[0, 0, 1] + aggp_ref[1, 0, 1]], axis=1)
        g = g_ref[0]
        hmix_ref[0] = g * rd[:, None] * agg + (1.0 - g) * h1_ref[0]

    return pl.pallas_call(
        body,
        grid=(R, N // NB),
        in_specs=[
            pl.BlockSpec(memory_space=pltpu.SMEM),
            pl.BlockSpec((NC, 1, 2, NB, H // 2), lambda r, n: (0, r, 0, n, 0)),
            pl.BlockSpec((1, NB, H), lambda r, n: (r, n, 0)),
            pl.BlockSpec((NC, 1, 2, NB, O), lambda r, n: (0, r, 0, n, 0)),
        ],
        out_specs=pl.BlockSpec((1, NB, H), lambda r, n: (r, n, 0)),
        out_shape=jax.ShapeDtypeStruct((R, N, H), f32),
    )(gamma1, aggp, h1, degp)


def _tc_bn_matmul2(hmix, bn1_scale, bn1_bias, W2, b2):
    def body(hmix_ref, sc_ref, bi_ref, w2_ref, b2_ref, h2_ref):
        h = hmix_ref[0]
        mu = jnp.mean(h, axis=0)
        var = jnp.mean((h - mu) ** 2, axis=0)
        hbn = (h - mu) * lax.rsqrt(var + 1e-5) * sc_ref[0, 0] + bi_ref[0, 0]
        hbn = jnp.maximum(hbn, 0.0)
        h2_ref[0] = lax.dot_general(hbn, w2_ref[0], (((1,), (0,)), ((), ())),
                                    preferred_element_type=f32) + b2_ref[0, 0]

    return pl.pallas_call(
        body,
        grid=(R,),
        in_specs=[
            pl.BlockSpec((1, N, H), lambda r: (r, 0, 0)),
            pl.BlockSpec((1, 1, H), lambda r: (r, 0, 0)),
            pl.BlockSpec((1, 1, H), lambda r: (r, 0, 0)),
            pl.BlockSpec((1, H, O), lambda r: (r, 0, 0)),
            pl.BlockSpec((1, 1, O), lambda r: (r, 0, 0)),
        ],
        out_specs=pl.BlockSpec((1, N, O), lambda r: (r, 0, 0)),
        out_shape=jax.ShapeDtypeStruct((R, N, O), f32),
    )(hmix, bn1_scale.reshape(R, 1, H), bn1_bias.reshape(R, 1, H), W2,
      b2.reshape(R, 1, O))


def _sc_edge_gather(h20, h21, h22, s0, d0, s1, d1, s2, d2):
    @functools.partial(
        pl.kernel,
        out_type=[
            jax.ShapeDtypeStruct((R, E, O), f32),
            jax.ShapeDtypeStruct((R, E, O), f32),
        ],
        mesh=_mesh(),
        scratch_types=[
            pltpu.VMEM((GPAD, K), i32),
            pltpu.VMEM((GPAD, K), i32),
            pltpu.VMEM((K, O), f32),
            pltpu.VMEM((K, O), f32),
            pltpu.VMEM((K, O), f32),
            pltpu.VMEM((K, O), f32),
            pltpu.SemaphoreType.DMA,
            pltpu.SemaphoreType.DMA,
            pltpu.SemaphoreType.DMA,
            pltpu.SemaphoreType.DMA,
        ],
        compiler_params=pltpu.CompilerParams(use_tc_tiling_on_sc=False),
    )
    def k(h20h, h21h, h22h, s0h, d0h, s1h, d1h, s2h, d2h, hs_out, hd_out,
          ixs, ixd, sA, dA, sB, dB, semSA, semDA, semSB, semDB):
        wid = _wid()
        h2s = [h20h, h21h, h22h]
        ess = [(s0h, d0h), (s1h, d1h), (s2h, d2h)]
        for r in range(R):
            sh, dh = ess[r]
            pltpu.sync_copy(sh.at[wid], ixs)
            pltpu.sync_copy(dh.at[wid], ixd)
            tbl = h2s[r]
            pltpu.async_copy(tbl.at[ixs.at[0]], sA, semSA)
            pltpu.async_copy(tbl.at[ixd.at[0]], dA, semDA)

            def body(t, _):
                j = 2 * t
                base = pl.multiple_of(wid * EPW + j * K, K)
                pltpu.async_copy(tbl.at[ixs.at[j + 1]], sB, semSB)
                pltpu.async_copy(tbl.at[ixd.at[j + 1]], dB, semDB)
                pltpu.make_async_copy(tbl.at[ixs.at[j]], sA, semSA).wait()
                pltpu.make_async_copy(tbl.at[ixd.at[j]], dA, semDA).wait()
                pltpu.sync_copy(sA, hs_out.at[r, pl.ds(base, K), :])
                pltpu.sync_copy(dA, hd_out.at[r, pl.ds(base, K), :])
                pltpu.async_copy(tbl.at[ixs.at[j + 2]], sA, semSA)
                pltpu.async_copy(tbl.at[ixd.at[j + 2]], dA, semDA)
                pltpu.make_async_copy(tbl.at[ixs.at[j + 1]], sB,
                                      semSB).wait()
                pltpu.make_async_copy(tbl.at[ixd.at[j + 1]], dB,
                                      semDB).wait()
                pltpu.sync_copy(sB, hs_out.at[r, pl.ds(base + K, K), :])
                pltpu.sync_copy(dB, hd_out.at[r, pl.ds(base + K, K), :])
                return 0

            lax.fori_loop(0, GMAX // 2, body, 0)
            lastb = pl.multiple_of(wid * EPW + (GMAX - 1) * K, K)
            pltpu.make_async_copy(tbl.at[ixs.at[GMAX - 1]], sA,
                                  semSA).wait()
            pltpu.make_async_copy(tbl.at[ixd.at[GMAX - 1]], dA,
                                  semDA).wait()
            pltpu.sync_copy(sA, hs_out.at[r, pl.ds(lastb, K), :])
            pltpu.sync_copy(dA, hd_out.at[r, pl.ds(lastb, K), :])

    return k(h20, h21, h22, s0, d0, s1, d1, s2, d2)


EB = 2000


def _tc_gaussian(hs, hd, gamma2):
    def body(g_ref, hs_ref, hd_ref, q_ref):
        df = hs_ref[0] - hd_ref[0]
        ss = jnp.sum(df * df, axis=1)
        w = jnp.exp(-g_ref[0] * ss)
        wb = jnp.broadcast_to(w[:, None], (EB, O))
        q_ref[0] = jnp.concatenate([hs_ref[0] * w[:, None], wb], axis=1)

    return pl.pallas_call(
        body,
        grid=(R, E // EB),
        in_specs=[
            pl.BlockSpec(memory_space=pltpu.SMEM),
            pl.BlockSpec((1, EB, O), lambda r, e: (r, e, 0)),
            pl.BlockSpec((1, EB, O), lambda r, e: (r, e, 0)),
        ],
        out_specs=pl.BlockSpec((1, EB, 2 * O), lambda r, e: (r, e, 0)),
        out_shape=jax.ShapeDtypeStruct((R, E, 2 * O), f32),
    )(gamma2, hs, hd)


def _sc_scatter2(q, d0, d1, d2, zNQ):
    @functools.partial(
        pl.kernel,
        out_type=jax.ShapeDtypeStruct((NC, R, NP, 2 * O), f32),
        mesh=_mesh(),
        scratch_types=[
            pltpu.VMEM_SHARED((NP, 2 * O), f32),
            pltpu.VMEM((GPAD, K), i32),
            pltpu.VMEM((K, 2 * O), f32),
            pltpu.VMEM((K, 2 * O), f32),
            pltpu.VMEM((640, 2 * O), f32),
            pltpu.SemaphoreType.DMA,
            pltpu.SemaphoreType.DMA,
        ],
        compiler_params=pltpu.CompilerParams(use_tc_tiling_on_sc=False),
    )
    def k(q_h, d0h, d1h, d2h, zNQ_h, accp, acc, ixd, rows, rowsB, zbufQ,
          semA, semB):
        c = lax.axis_index("c")
        s = lax.axis_index("s")
        wid = _wid()
        dhs = [d0h, d1h, d2h]
        pltpu.sync_copy(zNQ_h.at[pl.ds(0, 640), :], zbufQ)
        for r in range(R):
            pltpu.sync_copy(zbufQ, acc.at[pl.ds(s * 640, 640), :])
            pltpu.sync_copy(dhs[r].at[wid], ixd)
            plsc.subcore_barrier()
            base0 = pl.multiple_of(wid * EPW, K)
            pltpu.async_copy(q_h.at[r, pl.ds(base0, K), :], rows, semA)

            def body(t, _):
                j = 2 * t
                base = pl.multiple_of(wid * EPW + j * K, K)
                pltpu.async_copy(q_h.at[r, pl.ds(base + K, K), :], rowsB,
                                 semB)
                pltpu.make_async_copy(q_h.at[r, pl.ds(base, K), :], rows,
                                      semA).wait()
                pltpu.sync_copy(rows, acc.at[ixd.at[j]], add=True)
                pltpu.async_copy(q_h.at[r, pl.ds(base + 2 * K, K), :],
                                 rows, semA)
                pltpu.make_async_copy(q_h.at[r, pl.ds(base + K, K), :],
                                      rowsB, semB).wait()
                pltpu.sync_copy(rowsB, acc.at[ixd.at[j + 1]], add=True)
                return 0

            lax.fori_loop(0, GMAX // 2, body, 0)
            lastb = pl.multiple_of(wid * EPW + (GMAX - 1) * K, K)
            pltpu.make_async_copy(q_h.at[r, pl.ds(lastb, K), :], rows,
                                  semA).wait()
            pltpu.sync_copy(rows, acc.at[ixd.at[GMAX - 1]], add=True)
            plsc.subcore_barrier()
            pltpu.sync_copy(acc.at[pl.ds(s * 640, 640), :], zbufQ)
            pltpu.sync_copy(zbufQ, accp.at[c, r, pl.ds(s * 640, 640), :])
            pltpu.sync_copy(zNQ_h.at[pl.ds(0, 640), :], zbufQ)
            plsc.subcore_barrier()

    return k(q, d0, d1, d2, zNQ)


def _tc_norm_stats(accp):
    def body(accp_ref, hpre_ref, st_ref):
        q = accp_ref[0, 0, :N] + accp_ref[1, 0, :N]
        den = q[:, O]
        hpre = q[:, :O] / (den[:, None] + 1e-9)
        hpre_ref[0] = hpre
        mu = jnp.mean(hpre, axis=0)
        var = jnp.mean((hpre - mu) ** 2, axis=0)
        st_ref[0, 0] = mu
        st_ref[0, 1] = var

    return pl.pallas_call(
        body,
        grid=(R,),
        in_specs=[
            pl.BlockSpec((NC, 1, NP, 2 * O), lambda r: (0, r, 0, 0)),
        ],
        out_specs=[
            pl.BlockSpec((1, N, O), lambda r: (r, 0, 0)),
            pl.BlockSpec((1, 2, O), lambda r: (r, 0, 0)),
        ],
        out_shape=[
            jax.ShapeDtypeStruct((R, N, O), f32),
            jax.ShapeDtypeStruct((R, 2, O), f32),
        ],
    )(accp)


BPW = B // NW


def _sc_batch_gather(hp0, hp1, hp2, batch_nodes):
    @functools.partial(
        pl.kernel,
        out_type=jax.ShapeDtypeStruct((R, B, O), f32),
        mesh=_mesh(),
        scratch_types=[
            pltpu.VMEM((BPW,), i32),
            pltpu.VMEM((BPW, O), f32),
            pltpu.SemaphoreType.DMA,
        ],
        compiler_params=pltpu.CompilerParams(use_tc_tiling_on_sc=False),
    )
    def k(hp0h, hp1h, hp2h, bn_h, gb, ix, rows, sem):
        wid = _wid()
        hps = [hp0h, hp1h, hp2h]
        base = pl.multiple_of(wid * BPW, BPW)
        pltpu.sync_copy(bn_h.at[pl.ds(base, BPW)], ix)
        for r in range(R):
            pltpu.async_copy(hps[r].at[ix], rows, sem).wait()
            pltpu.sync_copy(rows, gb.at[r, pl.ds(base, BPW), :])

    return k(hp0, hp1, hp2, batch_nodes)


def _tc_final(gb, stats, bn2_scale, bn2_bias):
    def body(gb_ref, st_ref, sc_ref, bi_ref, out_ref):
        for r in range(R):
            x = gb_ref[r]
            mu = st_ref[r, 0]
            var = st_ref[r, 1]
            xb = (x - mu) * lax.rsqrt(var + 1e-5) * sc_ref[r] + bi_ref[r]
            xb = jnp.maximum(xb, 0.0)
            m = jnp.max(xb, axis=1, keepdims=True)
            lse = m + jnp.log(jnp.sum(jnp.exp(xb - m), axis=1,
                                      keepdims=True))
            out_ref[r] = xb - lse

    return pl.pallas_call(
        body,
        out_shape=jax.ShapeDtypeStruct((R, B, O), f32),
    )(gb, stats, bn2_scale, bn2_bias)


def kernel(features, edge_index_0, edge_index_1, edge_index_2, batch_nodes,
           W1, b1, gamma1, W2, b2, gamma2, bn1_scale, bn1_bias, bn2_scale,
           bn2_bias):
    onesK = jnp.ones((K, O), f32)
    zNP = jnp.zeros((640, O), f32)
    zNH = jnp.zeros((64, H // 2), f32)
    zNQ = jnp.zeros((640, 2 * O), f32)
    def _eshape(v):
        v = v.reshape(NW, GMAX, K)
        return jnp.pad(v, ((0, 0), (0, GPAD - GMAX), (0, 0)))
    s0, d0 = _eshape(edge_index_0[0]), _eshape(edge_index_0[1])
    s1, d1 = _eshape(edge_index_1[0]), _eshape(edge_index_1[1])
    s2, d2 = _eshape(edge_index_2[0]), _eshape(edge_index_2[1])
    degp = _sc_degrees(s0, d0, s1, d1, s2, d2,
                       onesK, zNP).reshape(NC, R, 2, NP, O)
    h1, h1s = _tc_matmul1(features, degp, W1, b1)
    hsplit = [[h1s[r, 0], h1s[r, 1]] for r in range(R)]
    aggp = _sc_segsum(hsplit, s0, d0, s1, d1, s2, d2, zNH)
    hmix = _tc_mix(aggp, h1, degp, gamma1)
    h2 = _tc_bn_matmul2(hmix, bn1_scale, bn1_bias, W2, b2)
    hs, hd = _sc_edge_gather(h2[0], h2[1], h2[2], s0, d0, s1, d1, s2, d2)
    q = _tc_gaussian(hs, hd, gamma2)
    accp = _sc_scatter2(q, d0, d1, d2, zNQ)
    hpre, stats = _tc_norm_stats(accp)
    gb = _sc_batch_gather(hpre[0], hpre[1], hpre[2], batch_nodes)
    outq = _tc_final(gb, stats, bn2_scale, bn2_bias)
    return jnp.transpose(outq, (1, 0, 2)).reshape(B, R * O)


# speedup vs baseline: 1.2718x; 1.2718x over previous
import functools

import jax
import jax.numpy as jnp
from jax import lax
from jax.experimental import pallas as pl
from jax.experimental.pallas import tpu as pltpu
from jax.experimental.pallas import tpu_sc as plsc

N = 10000
E = 320000
D = 128
H = 128
O = 16
R = 3
B = 4096

NP = 10240
NC = 2
NS = 16
NW = NC * NS
EPW = E // NW
K = 80
GMAX = EPW // K

_mesh = functools.partial(
    plsc.VectorSubcoreMesh, core_axis_name="c", subcore_axis_name="s",
    num_cores=NC, num_subcores=NS)

f32 = jnp.float32
i32 = jnp.int32


def _wid():
    return lax.axis_index("s") * NC + lax.axis_index("c")


def _sc_degrees(s0, d0, s1, d1, s2, d2, onesK, zNP):
    @functools.partial(
        pl.kernel,
        out_type=jax.ShapeDtypeStruct((NC, 6, NP, O), f32),
        mesh=_mesh(),
        scratch_types=[
            [pltpu.VMEM_SHARED((NP, O), f32) for _ in range(3)],
            [pltpu.VMEM((GMAX, K), i32) for _ in range(3)],
            pltpu.VMEM((K, O), f32),
            pltpu.VMEM((640, O), f32),
            [pltpu.SemaphoreType.DMA for _ in range(3)],
        ],
        compiler_params=pltpu.CompilerParams(use_tc_tiling_on_sc=False),
    )
    def k(s0h, d0h, s1h, d1h, s2h, d2h, ones_h, z_h, degp, accs, idxs,
          ones_v, buf_v, sems):
        c = lax.axis_index("c")
        s = lax.axis_index("s")
        wid = _wid()
        pltpu.sync_copy(ones_h, ones_v)
        pltpu.sync_copy(z_h.at[pl.ds(0, 640), :], buf_v)
        es = [s0h, d0h, s1h, d1h, s2h, d2h]
        for grp in range(2):
            for jj in range(3):
                pltpu.sync_copy(buf_v, accs[jj].at[pl.ds(s * 640, 640), :])
                pltpu.sync_copy(es[grp * 3 + jj].at[wid], idxs[jj])
            plsc.subcore_barrier()

            def body(g, _):
                cps = [pltpu.async_copy(ones_v,
                                        accs[jj].at[idxs[jj].at[g]],
                                        sems[jj], add=True)
                       for jj in range(3)]
                for cp in cps:
                    cp.wait()
                return 0

            lax.fori_loop(0, GMAX, body, 0)
            plsc.subcore_barrier()
            for jj in range(3):
                pltpu.sync_copy(accs[jj].at[pl.ds(s * 640, 640), :], buf_v)
                pltpu.sync_copy(buf_v,
                                degp.at[c, grp * 3 + jj,
                                        pl.ds(s * 640, 640), :])
            pltpu.sync_copy(z_h.at[pl.ds(0, 640), :], buf_v)
            plsc.subcore_barrier()

    return k(s0, d0, s1, d1, s2, d2, onesK, zNP)


def _tc_matmul1(features, degp, W1, b1):
    NB = 2000

    def body(x_ref, degp_ref, w_ref, b_ref, h1_ref, h1s_ref):
        ds = degp_ref[0, 0, 0, :, 0] + degp_ref[1, 0, 0, :, 0]
        rs = jnp.where(ds > 0, lax.rsqrt(jnp.maximum(ds, 1e-20)), 0.0)
        h1 = lax.dot_general(x_ref[...], w_ref[0],
                             (((1,), (0,)), ((), ())),
                             preferred_element_type=f32) + b_ref[0, 0]
        h1_ref[0] = h1
        h1s = h1 * rs[:, None]
        h1s_ref[0, 0] = h1s[:, :64]
        h1s_ref[0, 1] = h1s[:, 64:]

    return pl.pallas_call(
        body,
        grid=(R, N // NB),
        in_specs=[
            pl.BlockSpec((NB, D), lambda r, n: (n, 0)),
            pl.BlockSpec((NC, 1, 2, NB, O), lambda r, n: (0, r, 0, n, 0)),
            pl.BlockSpec((1, D, H), lambda r, n: (r, 0, 0)),
            pl.BlockSpec((1, 1, H), lambda r, n: (r, 0, 0)),
        ],
        out_specs=[
            pl.BlockSpec((1, NB, H), lambda r, n: (r, n, 0)),
            pl.BlockSpec((1, 2, NB, 64), lambda r, n: (r, 0, n, 0)),
        ],
        out_shape=[
            jax.ShapeDtypeStruct((R, N, H), f32),
            jax.ShapeDtypeStruct((R, 2, N, 64), f32),
        ],
    )(features, degp, W1, b1.reshape(R, 1, H))


def _sc_segsum(hsplit, s0, d0, s1, d1, s2, d2, zNH):
    HH = H // 2

    @functools.partial(
        pl.kernel,
        out_type=jax.ShapeDtypeStruct((NC, R, 2, NP, HH), f32),
        mesh=_mesh(),
        scratch_types=[
            pltpu.VMEM_SHARED((NP, HH), f32),
            pltpu.VMEM((GMAX, K), i32),
            pltpu.VMEM((GMAX, K), i32),
            pltpu.VMEM((K, HH), f32),
            pltpu.VMEM((K, HH), f32),
            pltpu.VMEM((64, HH), f32),
            pltpu.SemaphoreType.DMA,
            pltpu.SemaphoreType.DMA,
        ],
        compiler_params=pltpu.CompilerParams(use_tc_tiling_on_sc=False),
    )
    def k(hs00h, hs01h, hs10h, hs11h, hs20h, hs21h, s0h, d0h, s1h, d1h,
          s2h, d2h, z_h, aggp, acc, ixs, ixd, rows, rowsB, zbuf, semA,
          semB):
        c = lax.axis_index("c")
        s = lax.axis_index("s")
        wid = _wid()
        hss = [[hs00h, hs01h], [hs10h, hs11h], [hs20h, hs21h]]
        ess = [(s0h, d0h), (s1h, d1h), (s2h, d2h)]
        pltpu.sync_copy(z_h.at[pl.ds(0, 64), :], zbuf)
        for r in range(R):
            sh, dh = ess[r]
            pltpu.sync_copy(sh.at[wid], ixs)
            pltpu.sync_copy(dh.at[wid], ixd)
            for hh in range(2):
                for j in range(10):
                    pltpu.sync_copy(zbuf,
                                    acc.at[pl.ds(s * 640 + j * 64, 64), :])
                plsc.subcore_barrier()
                tbl = hss[r][hh]
                pltpu.async_copy(tbl.at[ixs.at[0]], rows, semA)

                def body(t, _):
                    j = 2 * t
                    pltpu.async_copy(tbl.at[ixs.at[j + 1]], rowsB, semB)
                    pltpu.make_async_copy(tbl.at[ixs.at[j]], rows,
                                          semA).wait()
                    pltpu.sync_copy(rows, acc.at[ixd.at[j]], add=True)
                    pltpu.async_copy(tbl.at[ixs.at[j + 2]], rows, semA)
                    pltpu.make_async_copy(tbl.at[ixs.at[j + 1]], rowsB,
                                          semB).wait()
                    pltpu.sync_copy(rowsB, acc.at[ixd.at[j + 1]], add=True)
                    return 0

                lax.fori_loop(0, GMAX // 2, body, 0)
                pltpu.make_async_copy(tbl.at[ixs.at[GMAX - 1]], rows,
                                      semA).wait()
                pltpu.sync_copy(rows, acc.at[ixd.at[GMAX - 1]], add=True)
                plsc.subcore_barrier()
                for j in range(8):
                    pltpu.sync_copy(acc.at[pl.ds(s * 640 + j * K, K), :],
                                    rows)
                    pltpu.sync_copy(
                        rows,
                        aggp.at[c, r, hh, pl.ds(s * 640 + j * K, K), :])
                plsc.subcore_barrier()

    return k(hsplit[0][0], hsplit[0][1], hsplit[1][0], hsplit[1][1],
             hsplit[2][0], hsplit[2][1], s0, d0, s1, d1, s2, d2, zNH)


def _tc_mix(aggp, h1, degp, gamma1):
    NB = 2000

    def body(g_ref, aggp_ref, h1_ref, degp_ref, hmix_ref):
        dd = degp_ref[0, 0, 1, :, 0] + degp_ref[1, 0, 1, :, 0]
        rd = jnp.where(dd > 0, lax.rsqrt(jnp.maximum(dd, 1e-20)), 0.0)
        agg = jnp.concatenate(
            [aggp_ref[0, 0, 0] + aggp_ref[1, 0, 0],
             aggp_ref[0, 0, 1] + aggp_ref[1, 0, 1]], axis=1)
        g = g_ref[0]
        hmix_ref[0] = g * rd[:, None] * agg + (1.0 - g) * h1_ref[0]

    return pl.pallas_call(
        body,
        grid=(R, N // NB),
        in_specs=[
            pl.BlockSpec(memory_space=pltpu.SMEM),
            pl.BlockSpec((NC, 1, 2, NB, H // 2), lambda r, n: (0, r, 0, n, 0)),
            pl.BlockSpec((1, NB, H), lambda r, n: (r, n, 0)),
            pl.BlockSpec((NC, 1, 2, NB, O), lambda r, n: (0, r, 0, n, 0)),
        ],
        out_specs=pl.BlockSpec((1, NB, H), lambda r, n: (r, n, 0)),
        out_shape=jax.ShapeDtypeStruct((R, N, H), f32),
    )(gamma1, aggp, h1, degp)


def _tc_bn_matmul2(hmix, bn1_scale, bn1_bias, W2, b2):
    def body(hmix_ref, sc_ref, bi_ref, w2_ref, b2_ref, h2_ref):
        h = hmix_ref[0]
        mu = jnp.mean(h, axis=0)
        var = jnp.mean((h - mu) ** 2, axis=0)
        hbn = (h - mu) * lax.rsqrt(var + 1e-5) * sc_ref[0, 0] + bi_ref[0, 0]
        hbn = jnp.maximum(hbn, 0.0)
        h2_ref[0] = lax.dot_general(hbn, w2_ref[0], (((1,), (0,)), ((), ())),
                                    preferred_element_type=f32) + b2_ref[0, 0]

    return pl.pallas_call(
        body,
        grid=(R,),
        in_specs=[
            pl.BlockSpec((1, N, H), lambda r: (r, 0, 0)),
            pl.BlockSpec((1, 1, H), lambda r: (r, 0, 0)),
            pl.BlockSpec((1, 1, H), lambda r: (r, 0, 0)),
            pl.BlockSpec((1, H, O), lambda r: (r, 0, 0)),
            pl.BlockSpec((1, 1, O), lambda r: (r, 0, 0)),
        ],
        out_specs=pl.BlockSpec((1, N, O), lambda r: (r, 0, 0)),
        out_shape=jax.ShapeDtypeStruct((R, N, O), f32),
    )(hmix, bn1_scale.reshape(R, 1, H), bn1_bias.reshape(R, 1, H), W2,
      b2.reshape(R, 1, O))


def _sc_edge_gather(h20, h21, h22, s0, d0, s1, d1, s2, d2):
    @functools.partial(
        pl.kernel,
        out_type=[
            jax.ShapeDtypeStruct((R, E, O), f32),
            jax.ShapeDtypeStruct((R, E, O), f32),
        ],
        mesh=_mesh(),
        scratch_types=[
            pltpu.VMEM((GMAX, K), i32),
            pltpu.VMEM((GMAX, K), i32),
            pltpu.VMEM((K, O), f32),
            pltpu.VMEM((K, O), f32),
            pltpu.VMEM((K, O), f32),
            pltpu.VMEM((K, O), f32),
            pltpu.SemaphoreType.DMA,
            pltpu.SemaphoreType.DMA,
            pltpu.SemaphoreType.DMA,
            pltpu.SemaphoreType.DMA,
        ],
        compiler_params=pltpu.CompilerParams(use_tc_tiling_on_sc=False),
    )
    def k(h20h, h21h, h22h, s0h, d0h, s1h, d1h, s2h, d2h, hs_out, hd_out,
          ixs, ixd, sA, dA, sB, dB, semSA, semDA, semSB, semDB):
        wid = _wid()
        h2s = [h20h, h21h, h22h]
        ess = [(s0h, d0h), (s1h, d1h), (s2h, d2h)]
        for r in range(R):
            sh, dh = ess[r]
            pltpu.sync_copy(sh.at[wid], ixs)
            pltpu.sync_copy(dh.at[wid], ixd)
            tbl = h2s[r]
            pltpu.async_copy(tbl.at[ixs.at[0]], sA, semSA)
            pltpu.async_copy(tbl.at[ixd.at[0]], dA, semDA)

            def body(t, _):
                j = 2 * t
                base = pl.multiple_of(wid * EPW + j * K, K)
                pltpu.async_copy(tbl.at[ixs.at[j + 1]], sB, semSB)
                pltpu.async_copy(tbl.at[ixd.at[j + 1]], dB, semDB)
                pltpu.make_async_copy(tbl.at[ixs.at[j]], sA, semSA).wait()
                pltpu.make_async_copy(tbl.at[ixd.at[j]], dA, semDA).wait()
                pltpu.sync_copy(sA, hs_out.at[r, pl.ds(base, K), :])
                pltpu.sync_copy(dA, hd_out.at[r, pl.ds(base, K), :])
                pltpu.async_copy(tbl.at[ixs.at[j + 2]], sA, semSA)
                pltpu.async_copy(tbl.at[ixd.at[j + 2]], dA, semDA)
                pltpu.make_async_copy(tbl.at[ixs.at[j + 1]], sB,
                                      semSB).wait()
                pltpu.make_async_copy(tbl.at[ixd.at[j + 1]], dB,
                                      semDB).wait()
                pltpu.sync_copy(sB, hs_out.at[r, pl.ds(base + K, K), :])
                pltpu.sync_copy(dB, hd_out.at[r, pl.ds(base + K, K), :])
                return 0

            lax.fori_loop(0, GMAX // 2, body, 0)
            lastb = pl.multiple_of(wid * EPW + (GMAX - 1) * K, K)
            pltpu.make_async_copy(tbl.at[ixs.at[GMAX - 1]], sA,
                                  semSA).wait()
            pltpu.make_async_copy(tbl.at[ixd.at[GMAX - 1]], dA,
                                  semDA).wait()
            pltpu.sync_copy(sA, hs_out.at[r, pl.ds(lastb, K), :])
            pltpu.sync_copy(dA, hd_out.at[r, pl.ds(lastb, K), :])

    return k(h20, h21, h22, s0, d0, s1, d1, s2, d2)


EB = 2000


def _tc_gaussian(hs, hd, gamma2):
    def body(g_ref, hs_ref, hd_ref, q_ref):
        df = hs_ref[0] - hd_ref[0]
        ss = jnp.sum(df * df, axis=1)
        w = jnp.exp(-g_ref[0] * ss)
        wb = jnp.broadcast_to(w[:, None], (EB, O))
        q_ref[0] = jnp.concatenate([hs_ref[0] * w[:, None], wb], axis=1)

    return pl.pallas_call(
        body,
        grid=(R, E // EB),
        in_specs=[
            pl.BlockSpec(memory_space=pltpu.SMEM),
            pl.BlockSpec((1, EB, O), lambda r, e: (r, e, 0)),
            pl.BlockSpec((1, EB, O), lambda r, e: (r, e, 0)),
        ],
        out_specs=pl.BlockSpec((1, EB, 2 * O), lambda r, e: (r, e, 0)),
        out_shape=jax.ShapeDtypeStruct((R, E, 2 * O), f32),
    )(gamma2, hs, hd)


def _sc_scatter2(q, d0, d1, d2, zNQ):
    @functools.partial(
        pl.kernel,
        out_type=jax.ShapeDtypeStruct((NC, R, NP, 2 * O), f32),
        mesh=_mesh(),
        scratch_types=[
            pltpu.VMEM_SHARED((NP, 2 * O), f32),
            pltpu.VMEM((GMAX, K), i32),
            pltpu.VMEM((K, 2 * O), f32),
            pltpu.VMEM((K, 2 * O), f32),
            pltpu.VMEM((640, 2 * O), f32),
            pltpu.SemaphoreType.DMA,
            pltpu.SemaphoreType.DMA,
        ],
        compiler_params=pltpu.CompilerParams(use_tc_tiling_on_sc=False),
    )
    def k(q_h, d0h, d1h, d2h, zNQ_h, accp, acc, ixd, rows, rowsB, zbufQ,
          semA, semB):
        c = lax.axis_index("c")
        s = lax.axis_index("s")
        wid = _wid()
        dhs = [d0h, d1h, d2h]
        pltpu.sync_copy(zNQ_h.at[pl.ds(0, 640), :], zbufQ)
        for r in range(R):
            pltpu.sync_copy(zbufQ, acc.at[pl.ds(s * 640, 640), :])
            pltpu.sync_copy(dhs[r].at[wid], ixd)
            plsc.subcore_barrier()
            base0 = pl.multiple_of(wid * EPW, K)
            pltpu.async_copy(q_h.at[r, pl.ds(base0, K), :], rows, semA)

            def body(t, _):
                j = 2 * t
                base = pl.multiple_of(wid * EPW + j * K, K)
                pltpu.async_copy(q_h.at[r, pl.ds(base + K, K), :], rowsB,
                                 semB)
                pltpu.make_async_copy(q_h.at[r, pl.ds(base, K), :], rows,
                                      semA).wait()
                pltpu.sync_copy(rows, acc.at[ixd.at[j]], add=True)
                pltpu.async_copy(q_h.at[r, pl.ds(base + 2 * K, K), :],
                                 rows, semA)
                pltpu.make_async_copy(q_h.at[r, pl.ds(base + K, K), :],
                                      rowsB, semB).wait()
                pltpu.sync_copy(rowsB, acc.at[ixd.at[j + 1]], add=True)
                return 0

            lax.fori_loop(0, GMAX // 2, body, 0)
            lastb = pl.multiple_of(wid * EPW + (GMAX - 1) * K, K)
            pltpu.make_async_copy(q_h.at[r, pl.ds(lastb, K), :], rows,
                                  semA).wait()
            pltpu.sync_copy(rows, acc.at[ixd.at[GMAX - 1]], add=True)
            plsc.subcore_barrier()
            pltpu.sync_copy(acc.at[pl.ds(s * 640, 640), :], zbufQ)
            pltpu.sync_copy(zbufQ, accp.at[c, r, pl.ds(s * 640, 640), :])
            pltpu.sync_copy(zNQ_h.at[pl.ds(0, 640), :], zbufQ)
            plsc.subcore_barrier()

    return k(q, d0, d1, d2, zNQ)


def _tc_norm_stats(accp):
    def body(accp_ref, hpre_ref, st_ref):
        q = accp_ref[0, 0, :N] + accp_ref[1, 0, :N]
        den = q[:, O]
        hpre = q[:, :O] / (den[:, None] + 1e-9)
        hpre_ref[0] = hpre
        mu = jnp.mean(hpre, axis=0)
        var = jnp.mean((hpre - mu) ** 2, axis=0)
        st_ref[0, 0] = mu
        st_ref[0, 1] = var

    return pl.pallas_call(
        body,
        grid=(R,),
        in_specs=[
            pl.BlockSpec((NC, 1, NP, 2 * O), lambda r: (0, r, 0, 0)),
        ],
        out_specs=[
            pl.BlockSpec((1, N, O), lambda r: (r, 0, 0)),
            pl.BlockSpec((1, 2, O), lambda r: (r, 0, 0)),
        ],
        out_shape=[
            jax.ShapeDtypeStruct((R, N, O), f32),
            jax.ShapeDtypeStruct((R, 2, O), f32),
        ],
    )(accp)


BPW = B // NW


def _sc_batch_gather(hp0, hp1, hp2, batch_nodes):
    @functools.partial(
        pl.kernel,
        out_type=jax.ShapeDtypeStruct((R, B, O), f32),
        mesh=_mesh(),
        scratch_types=[
            pltpu.VMEM((BPW,), i32),
            pltpu.VMEM((BPW, O), f32),
            pltpu.SemaphoreType.DMA,
        ],
        compiler_params=pltpu.CompilerParams(use_tc_tiling_on_sc=False),
    )
    def k(hp0h, hp1h, hp2h, bn_h, gb, ix, rows, sem):
        wid = _wid()
        hps = [hp0h, hp1h, hp2h]
        base = pl.multiple_of(wid * BPW, BPW)
        pltpu.sync_copy(bn_h.at[pl.ds(base, BPW)], ix)
        for r in range(R):
            pltpu.async_copy(hps[r].at[ix], rows, sem).wait()
            pltpu.sync_copy(rows, gb.at[r, pl.ds(base, BPW), :])

    return k(hp0, hp1, hp2, batch_nodes)


def _tc_final(gb, stats, bn2_scale, bn2_bias):
    def body(gb_ref, st_ref, sc_ref, bi_ref, out_ref):
        for r in range(R):
            x = gb_ref[r]
            mu = st_ref[r, 0]
            var = st_ref[r, 1]
            xb = (x - mu) * lax.rsqrt(var + 1e-5) * sc_ref[r] + bi_ref[r]
            xb = jnp.maximum(xb, 0.0)
            m = jnp.max(xb, axis=1, keepdims=True)
            lse = m + jnp.log(jnp.sum(jnp.exp(xb - m), axis=1,
                                      keepdims=True))
            out_ref[r] = xb - lse

    return pl.pallas_call(
        body,
        out_shape=jax.ShapeDtypeStruct((R, B, O), f32),
    )(gb, stats, bn2_scale, bn2_bias)


def kernel(features, edge_index_0, edge_index_1, edge_index_2, batch_nodes,
           W1, b1, gamma1, W2, b2, gamma2, bn1_scale, bn1_bias, bn2_scale,
           bn2_bias):
    onesK = jnp.ones((K, O), f32)
    zNP = jnp.zeros((640, O), f32)
    zNH = jnp.zeros((64, H // 2), f32)
    zNQ = jnp.zeros((640, 2 * O), f32)
    s0, d0 = edge_index_0[0].reshape(NW, GMAX, K), edge_index_0[1].reshape(NW, GMAX, K)
    s1, d1 = edge_index_1[0].reshape(NW, GMAX, K), edge_index_1[1].reshape(NW, GMAX, K)
    s2, d2 = edge_index_2[0].reshape(NW, GMAX, K), edge_index_2[1].reshape(NW, GMAX, K)
    degp = _sc_degrees(s0, d0, s1, d1, s2, d2,
                       onesK, zNP).reshape(NC, R, 2, NP, O)
    h1, h1s = _tc_matmul1(features, degp, W1, b1)
    hsplit = [[h1s[r, 0], h1s[r, 1]] for r in range(R)]
    aggp = _sc_segsum(hsplit, s0, d0, s1, d1, s2, d2, zNH)
    hmix = _tc_mix(aggp, h1, degp, gamma1)
    h2 = _tc_bn_matmul2(hmix, bn1_scale, bn1_bias, W2, b2)
    hs, hd = _sc_edge_gather(h2[0], h2[1], h2[2], s0, d0, s1, d1, s2, d2)
    q = _tc_gaussian(hs, hd, gamma2)
    accp = _sc_scatter2(q, d0, d1, d2, zNQ)
    hpre, stats = _tc_norm_stats(accp)
    gb = _sc_batch_gather(hpre[0], hpre[1], hpre[2], batch_nodes)
    outq = _tc_final(gb, stats, bn2_scale, bn2_bias)
    return jnp.transpose(outq, (1, 0, 2)).reshape(B, R * O)
